# Initial kernel scaffold; baseline (speedup 1.0000x reference)
import jax
import jax.numpy as jnp
from jax.experimental import pallas as pl


def kernel(z, W, b):
    raise NotImplementedError("write your pallas kernel here")


# speedup vs baseline: 3.5196x; 3.5196x over previous
import functools
import jax
import jax.numpy as jnp
from jax.experimental import pallas as pl

_M = 30
_ENT_Q = 0.5
_NBINS = 256


def _k1_body(z_ref, wt_ref, b_ref, ent_ref, y_ref, zn_ref):
    z = z_ref[...]
    l = jnp.dot(z, wt_ref[...], preferred_element_type=jnp.float32) + b_ref[...]
    m = jnp.max(l, axis=1, keepdims=True)
    e = jnp.exp(l - m)
    s = jnp.sum(e, axis=1)
    t = jnp.sum(e * (l - m), axis=1)
    ent = jnp.log(s) - t / s
    idx = jax.lax.broadcasted_iota(jnp.int32, l.shape, 1)
    y = jnp.min(jnp.where(l >= m, idx, jnp.int32(1 << 30)), axis=1)
    ent_ref[...] = ent.reshape(ent_ref.shape)
    y_ref[...] = y.reshape(y_ref.shape)
    nn = jnp.sqrt(jnp.sum(z * z, axis=1, keepdims=True))
    zn_ref[...] = z / jnp.maximum(nn, 1e-12)


def _k2_body(ent_ref, y_ref, zn_ref, w_ref, c_ref, *, k_keep, kp, nrows, ncols):
    ent = ent_ref[...]
    y = y_ref[...]
    bits = jax.lax.bitcast_convert_type(ent, jnp.int32)
    key = jnp.where(bits < 0, bits ^ jnp.int32(0x7FFFFFFF), bits)

    bin_col = jax.lax.broadcasted_iota(jnp.int32, (_NBINS, 1), 0)
    cls_col = jax.lax.broadcasted_iota(jnp.int32, (kp, 1), 0)
    bin_row = jax.lax.broadcasted_iota(jnp.int32, (1, _NBINS), 1)
    row_col = jax.lax.broadcasted_iota(jnp.int32, (nrows, 1), 0)

    def _prefix_lanes(x):
        lane = jax.lax.broadcasted_iota(jnp.int32, x.shape, 1)
        s = 1
        while s < x.shape[-1]:
            x = x + jnp.where(lane >= s, jnp.roll(x, s, axis=1), 0.0)
            s *= 2
        return x

    def _row(a, i):
        zero = jnp.zeros((), a.dtype)
        return jnp.sum(jnp.where(row_col == i, a, zero), axis=0, keepdims=True)

    def _gather_by_class(vals):
        def body(i, acc):
            at = jnp.where(cls_col == _row(y, i), 1.0, 0.0)
            srow = jnp.sum(at * vals, axis=0, keepdims=True)
            return jnp.where(row_col == i, srow, acc)
        return jax.lax.fori_loop(0, nrows, body, jnp.zeros((nrows, ncols), jnp.float32))

    matched = jnp.ones((nrows, ncols), jnp.float32)
    target = jnp.float32(k_keep)
    thresh = jnp.int32(0)
    for p in range(4):
        sh = 24 - 8 * p
        digit = (key >> sh) & 0xFF

        def hbody(i, hist):
            dt = jnp.where(bin_col == _row(digit, i), 1.0, 0.0)
            mrow = _row(matched, i)
            return hist + jax.lax.dot_general(
                mrow.astype(jnp.bfloat16), dt.astype(jnp.bfloat16),
                dimension_numbers=(((1,), (1,)), ((), ())),
                preferred_element_type=jnp.float32)
        hist = jax.lax.fori_loop(0, nrows, hbody, jnp.zeros((1, _NBINS), jnp.float32))
        cum = _prefix_lanes(hist)
        c = jnp.sum((cum < target).astype(jnp.int32))
        cum_ex = cum - hist
        target = target - jnp.sum(jnp.where(bin_row == c, cum_ex, 0.0))
        matched = matched * (digit == c).astype(jnp.float32)
        thresh = thresh | (c << sh)
    keep = (key <= thresh).astype(jnp.float32)

    def cbody(i, cnt):
        at = jnp.where(cls_col == _row(y, i), _row(keep, i), 0.0)
        return cnt + jnp.sum(at, axis=1, keepdims=True)
    cnt = jax.lax.fori_loop(0, nrows, cbody, jnp.zeros((kp, 1), jnp.float32))

    needs = jnp.full((kp, 1), float(_M), jnp.float32)
    matched2 = keep
    lessf = jnp.zeros((nrows, ncols), jnp.float32)
    for p in range(4):
        sh = 24 - 8 * p
        digit = (key >> sh) & 0xFF

        def hmbody(i, hmat):
            at = jnp.where(cls_col == _row(y, i), _row(matched2, i), 0.0)
            dt = jnp.where(bin_col == _row(digit, i), 1.0, 0.0)
            return hmat + jax.lax.dot_general(
                at.astype(jnp.bfloat16), dt.astype(jnp.bfloat16),
                dimension_numbers=(((1,), (1,)), ((), ())),
                preferred_element_type=jnp.float32)
        hmat = jax.lax.fori_loop(0, nrows, hmbody,
                                 jnp.zeros((kp, _NBINS), jnp.float32))
        cum = _prefix_lanes(hmat)
        c_k = jnp.sum((cum < needs).astype(jnp.int32), axis=1, keepdims=True)
        c_cl = jnp.minimum(c_k, _NBINS - 1)
        cum_ex = cum - hmat
        sel = jnp.sum(jnp.where(bin_row == c_cl, cum_ex, 0.0), axis=1, keepdims=True)
        needs = needs - sel
        c_y = _gather_by_class(c_cl.astype(jnp.float32))
        d_f = digit.astype(jnp.float32)
        lessf = lessf + matched2 * (d_f < c_y)
        matched2 = matched2 * (d_f == c_y).astype(jnp.float32)

    cnt_small = (cnt < float(_M)).astype(jnp.float32)
    cy_small = _gather_by_class(cnt_small)
    final = keep * jnp.maximum(cy_small, jnp.minimum(lessf + matched2, 1.0))

    d = zn_ref.shape[1]

    def sbody(i, sums):
        atw = jnp.where(cls_col == _row(y, i), _row(final, i), 0.0)
        zchunk = zn_ref[pl.ds(i * ncols, ncols), :]
        return sums + jnp.dot(atw, zchunk, preferred_element_type=jnp.float32)
    sums = jax.lax.fori_loop(0, nrows, sbody, jnp.zeros((kp, d), jnp.float32))

    wmat = w_ref[...]
    k_real = wmat.shape[0]
    wn = jnp.sqrt(jnp.sum(wmat * wmat, axis=1, keepdims=True))
    anchor = wmat / jnp.maximum(wn, 1e-12)
    counts = jnp.minimum(cnt[:k_real, :], float(_M))
    cmat = (anchor + sums[:k_real, :]) / (1.0 + counts)
    cn = jnp.sqrt(jnp.sum(cmat * cmat, axis=1, keepdims=True))
    c_ref[...] = cmat / jnp.maximum(cn, 1e-12)


def _k3_body(zn_ref, c_ref, b_ref, out_ref):
    out_ref[...] = jax.lax.dot_general(
        zn_ref[...], c_ref[...],
        dimension_numbers=(((1,), (1,)), ((), ())),
        preferred_element_type=jnp.float32) + b_ref[...]


@jax.jit
def kernel(z, W, b):
    B, D = z.shape
    K = W.shape[0]
    TB = 512
    nt = B // TB
    ncols = 2048
    nrows = B // ncols
    kp = 1024
    k_keep = max(1, int(round(B * _ENT_Q)))

    b2 = b.reshape(1, K)
    wt = W.T

    ent, y, zn = pl.pallas_call(
        _k1_body,
        grid=(nt,),
        in_specs=[
            pl.BlockSpec((TB, D), lambda i: (i, 0)),
            pl.BlockSpec((D, K), lambda i: (0, 0)),
            pl.BlockSpec((1, K), lambda i: (0, 0)),
        ],
        out_specs=[
            pl.BlockSpec((1, 1, TB), lambda i: (i, 0, 0)),
            pl.BlockSpec((1, 1, TB), lambda i: (i, 0, 0)),
            pl.BlockSpec((TB, D), lambda i: (i, 0)),
        ],
        out_shape=[
            jax.ShapeDtypeStruct((nt, 1, TB), jnp.float32),
            jax.ShapeDtypeStruct((nt, 1, TB), jnp.int32),
            jax.ShapeDtypeStruct((B, D), jnp.float32),
        ],
    )(z, wt, b2)

    ent2 = ent.reshape(nrows, ncols)
    y2 = y.reshape(nrows, ncols)

    cmat = pl.pallas_call(
        functools.partial(_k2_body, k_keep=k_keep, kp=kp, nrows=nrows, ncols=ncols),
        grid=(1,),
        in_specs=[
            pl.BlockSpec((nrows, ncols), lambda i: (0, 0)),
            pl.BlockSpec((nrows, ncols), lambda i: (0, 0)),
            pl.BlockSpec((B, D), lambda i: (0, 0)),
            pl.BlockSpec((K, D), lambda i: (0, 0)),
        ],
        out_specs=pl.BlockSpec((K, D), lambda i: (0, 0)),
        out_shape=jax.ShapeDtypeStruct((K, D), jnp.float32),
    )(ent2, y2, zn, W)

    out = pl.pallas_call(
        _k3_body,
        grid=(nt,),
        in_specs=[
            pl.BlockSpec((TB, D), lambda i: (i, 0)),
            pl.BlockSpec((K, D), lambda i: (0, 0)),
            pl.BlockSpec((1, K), lambda i: (0, 0)),
        ],
        out_specs=pl.BlockSpec((TB, K), lambda i: (i, 0)),
        out_shape=jax.ShapeDtypeStruct((B, K), jnp.float32),
    )(zn, cmat, b2)
    return out
